# Initial kernel scaffold; baseline (speedup 1.0000x reference)
import jax
import jax.numpy as jnp
from jax.experimental import pallas as pl


def kernel(hidden_states, weight, e_score_correction_bias):
    raise NotImplementedError("write your pallas kernel here")


# speedup vs baseline: 3.1902x; 3.1902x over previous
import jax
import jax.numpy as jnp
from jax import lax
from jax.experimental import pallas as pl

HIDDEN = 2048
N_EXPERTS = 64
TOP_K = 8
N_GROUP = 8
GSIZE = N_EXPERTS // N_GROUP
TOPK_GROUP = 4
SCALE = 2.5
TB = 512


def _router_body(hs_ref, wt_ref, b_ref, idx_ref, w_ref):
    logits = jnp.dot(hs_ref[...], wt_ref[...], preferred_element_type=jnp.float32)
    scores = jax.nn.sigmoid(logits)
    sfc = scores + b_ref[...]
    stf = sfc.T
    sraw_t = scores.T

    g = stf.reshape(N_GROUP, GSIZE, TB)
    m1 = jnp.max(g, axis=1)
    io8 = lax.broadcasted_iota(jnp.int32, (N_GROUP, GSIZE, TB), 1)
    i1 = jnp.min(jnp.where(g == m1[:, None, :], io8, GSIZE), axis=1)
    g2 = jnp.where(io8 == i1[:, None, :], -jnp.inf, g)
    m2 = jnp.max(g2, axis=1)
    gs = m1 + m2

    giota = lax.broadcasted_iota(jnp.int32, (N_GROUP, TB), 0)
    gmask = jnp.zeros((N_GROUP, TB), jnp.bool_)
    work = gs
    for _ in range(TOPK_GROUP):
        mg = jnp.max(work, axis=0)
        gi = jnp.min(jnp.where(work == mg[None, :], giota, N_GROUP), axis=0)
        sel = giota == gi[None, :]
        gmask = jnp.logical_or(gmask, sel)
        work = jnp.where(sel, -jnp.inf, work)

    emask = jnp.broadcast_to(gmask[:, None, :], (N_GROUP, GSIZE, TB))
    emask = emask.reshape(N_EXPERTS, TB)
    masked = jnp.where(emask, stf, 0.0)

    eiota = lax.broadcasted_iota(jnp.int32, (N_EXPERTS, TB), 0)
    idxs = []
    vals = []
    m = masked
    for _ in range(TOP_K):
        mv = jnp.max(m, axis=0)
        ei = jnp.min(jnp.where(m == mv[None, :], eiota, N_EXPERTS), axis=0)
        sel = eiota == ei[None, :]
        val = jnp.max(jnp.where(sel, sraw_t, -jnp.inf), axis=0)
        idxs.append(ei)
        vals.append(val)
        m = jnp.where(sel, -1.0, m)

    idx_mat = jnp.stack(idxs, axis=0)
    w_mat = jnp.stack(vals, axis=0)
    denom = jnp.sum(w_mat, axis=0, keepdims=True) + 1e-20
    w_ref[...] = w_mat / denom * SCALE
    idx_ref[...] = idx_mat


def kernel(hidden_states, weight, e_score_correction_bias):
    tokens = hidden_states.shape[0]
    hs = hidden_states.reshape(tokens, HIDDEN).astype(jnp.float32)
    wt = weight.astype(jnp.float32).T
    bias = e_score_correction_bias.reshape(1, N_EXPERTS).astype(jnp.float32)

    grid = (tokens // TB,)
    idx_t, w_t = pl.pallas_call(
        _router_body,
        grid=grid,
        in_specs=[
            pl.BlockSpec((TB, HIDDEN), lambda i: (i, 0)),
            pl.BlockSpec((HIDDEN, N_EXPERTS), lambda i: (0, 0)),
            pl.BlockSpec((1, N_EXPERTS), lambda i: (0, 0)),
        ],
        out_specs=[
            pl.BlockSpec((TOP_K, TB), lambda i: (0, i)),
            pl.BlockSpec((TOP_K, TB), lambda i: (0, i)),
        ],
        out_shape=[
            jax.ShapeDtypeStruct((TOP_K, tokens), jnp.int32),
            jax.ShapeDtypeStruct((TOP_K, tokens), jnp.float32),
        ],
    )(hs, wt, bias)
    return idx_t.T, w_t.T


# speedup vs baseline: 3.7450x; 1.1739x over previous
import jax
import jax.numpy as jnp
from jax import lax
from jax.experimental import pallas as pl

HIDDEN = 2048
N_EXPERTS = 64
TOP_K = 8
N_GROUP = 8
GSIZE = N_EXPERTS // N_GROUP
TOPK_GROUP = 4
SCALE = 2.5
TB = 1024


def _router_body(hs_ref, wt_ref, b_ref, idx_ref, w_ref):
    logits = jnp.dot(hs_ref[...], wt_ref[...], preferred_element_type=jnp.float32)
    scores = jax.nn.sigmoid(logits)
    sfc = scores + b_ref[...]
    stf = sfc.T
    sraw_t = scores.T

    g = stf.reshape(N_GROUP, GSIZE, TB)
    m1 = jnp.max(g, axis=1)
    io8 = lax.broadcasted_iota(jnp.int32, (N_GROUP, GSIZE, TB), 1)
    i1 = jnp.min(jnp.where(g == m1[:, None, :], io8, GSIZE), axis=1)
    g2 = jnp.where(io8 == i1[:, None, :], -jnp.inf, g)
    m2 = jnp.max(g2, axis=1)
    gs = m1 + m2

    giota = lax.broadcasted_iota(jnp.int32, (N_GROUP, TB), 0)
    gmask = jnp.zeros((N_GROUP, TB), jnp.bool_)
    work = gs
    for _ in range(TOPK_GROUP):
        mg = jnp.max(work, axis=0)
        gi = jnp.min(jnp.where(work == mg[None, :], giota, N_GROUP), axis=0)
        sel = giota == gi[None, :]
        gmask = jnp.logical_or(gmask, sel)
        work = jnp.where(sel, -jnp.inf, work)

    emask = jnp.broadcast_to(gmask[:, None, :], (N_GROUP, GSIZE, TB))
    emask = emask.reshape(N_EXPERTS, TB)
    masked = jnp.where(emask, stf, 0.0)

    eiota = lax.broadcasted_iota(jnp.int32, (N_EXPERTS, TB), 0)
    idxs = []
    vals = []
    m = masked
    for _ in range(TOP_K):
        mv = jnp.max(m, axis=0)
        ei = jnp.min(jnp.where(m == mv[None, :], eiota, N_EXPERTS), axis=0)
        sel = eiota == ei[None, :]
        val = jnp.max(jnp.where(sel, sraw_t, -jnp.inf), axis=0)
        idxs.append(ei)
        vals.append(val)
        m = jnp.where(sel, -1.0, m)

    idx_mat = jnp.stack(idxs, axis=0)
    w_mat = jnp.stack(vals, axis=0)
    denom = jnp.sum(w_mat, axis=0, keepdims=True) + 1e-20
    w_ref[...] = w_mat / denom * SCALE
    idx_ref[...] = idx_mat


def kernel(hidden_states, weight, e_score_correction_bias):
    tokens = hidden_states.shape[0]
    hs = hidden_states.reshape(tokens, HIDDEN).astype(jnp.float32)
    wt = weight.astype(jnp.float32).T
    bias = e_score_correction_bias.reshape(1, N_EXPERTS).astype(jnp.float32)

    grid = (tokens // TB,)
    idx_t, w_t = pl.pallas_call(
        _router_body,
        grid=grid,
        in_specs=[
            pl.BlockSpec((TB, HIDDEN), lambda i: (i, 0)),
            pl.BlockSpec((HIDDEN, N_EXPERTS), lambda i: (0, 0)),
            pl.BlockSpec((1, N_EXPERTS), lambda i: (0, 0)),
        ],
        out_specs=[
            pl.BlockSpec((TOP_K, TB), lambda i: (0, i)),
            pl.BlockSpec((TOP_K, TB), lambda i: (0, i)),
        ],
        out_shape=[
            jax.ShapeDtypeStruct((TOP_K, tokens), jnp.int32),
            jax.ShapeDtypeStruct((TOP_K, tokens), jnp.float32),
        ],
    )(hs, wt, bias)
    return idx_t.T, w_t.T


# speedup vs baseline: 3.8684x; 1.0330x over previous
import jax
import jax.numpy as jnp
from jax import lax
from jax.experimental import pallas as pl

HIDDEN = 2048
N_EXPERTS = 64
TOP_K = 8
N_GROUP = 8
GSIZE = N_EXPERTS // N_GROUP
TOPK_GROUP = 4
SCALE = 2.5
TB = 2048


def _router_body(hs_ref, wt_ref, b_ref, idx_ref, w_ref):
    logits = jnp.dot(hs_ref[...], wt_ref[...], preferred_element_type=jnp.float32)
    scores = jax.nn.sigmoid(logits)
    sfc = scores + b_ref[...]
    stf = sfc.T
    sraw_t = scores.T

    g = stf.reshape(N_GROUP, GSIZE, TB)
    m1 = jnp.max(g, axis=1)
    io8 = lax.broadcasted_iota(jnp.int32, (N_GROUP, GSIZE, TB), 1)
    i1 = jnp.min(jnp.where(g == m1[:, None, :], io8, GSIZE), axis=1)
    g2 = jnp.where(io8 == i1[:, None, :], -jnp.inf, g)
    m2 = jnp.max(g2, axis=1)
    gs = m1 + m2

    giota = lax.broadcasted_iota(jnp.int32, (N_GROUP, TB), 0)
    gmask = jnp.zeros((N_GROUP, TB), jnp.bool_)
    work = gs
    for _ in range(TOPK_GROUP):
        mg = jnp.max(work, axis=0)
        gi = jnp.min(jnp.where(work == mg[None, :], giota, N_GROUP), axis=0)
        sel = giota == gi[None, :]
        gmask = jnp.logical_or(gmask, sel)
        work = jnp.where(sel, -jnp.inf, work)

    emask = jnp.broadcast_to(gmask[:, None, :], (N_GROUP, GSIZE, TB))
    emask = emask.reshape(N_EXPERTS, TB)
    masked = jnp.where(emask, stf, 0.0)

    eiota = lax.broadcasted_iota(jnp.int32, (N_EXPERTS, TB), 0)
    idxs = []
    vals = []
    m = masked
    for _ in range(TOP_K):
        mv = jnp.max(m, axis=0)
        ei = jnp.min(jnp.where(m == mv[None, :], eiota, N_EXPERTS), axis=0)
        sel = eiota == ei[None, :]
        val = jnp.max(jnp.where(sel, sraw_t, -jnp.inf), axis=0)
        idxs.append(ei)
        vals.append(val)
        m = jnp.where(sel, -1.0, m)

    idx_mat = jnp.stack(idxs, axis=0)
    w_mat = jnp.stack(vals, axis=0)
    denom = jnp.sum(w_mat, axis=0, keepdims=True) + 1e-20
    w_ref[...] = w_mat / denom * SCALE
    idx_ref[...] = idx_mat


def kernel(hidden_states, weight, e_score_correction_bias):
    tokens = hidden_states.shape[0]
    hs = hidden_states.reshape(tokens, HIDDEN).astype(jnp.float32)
    wt = weight.astype(jnp.float32).T
    bias = e_score_correction_bias.reshape(1, N_EXPERTS).astype(jnp.float32)

    grid = (tokens // TB,)
    idx_t, w_t = pl.pallas_call(
        _router_body,
        grid=grid,
        in_specs=[
            pl.BlockSpec((TB, HIDDEN), lambda i: (i, 0)),
            pl.BlockSpec((HIDDEN, N_EXPERTS), lambda i: (0, 0)),
            pl.BlockSpec((1, N_EXPERTS), lambda i: (0, 0)),
        ],
        out_specs=[
            pl.BlockSpec((TOP_K, TB), lambda i: (0, i)),
            pl.BlockSpec((TOP_K, TB), lambda i: (0, i)),
        ],
        out_shape=[
            jax.ShapeDtypeStruct((TOP_K, tokens), jnp.int32),
            jax.ShapeDtypeStruct((TOP_K, tokens), jnp.float32),
        ],
    )(hs, wt, bias)
    return idx_t.T, w_t.T
